# Initial kernel scaffold; baseline (speedup 1.0000x reference)
import jax
import jax.numpy as jnp
from jax.experimental import pallas as pl


def kernel(x_tasks, x_data, edge_index_dt, edge_index_td, edge_attr_dt, edge_attr_td, params):
    raise NotImplementedError("write your pallas kernel here")


# speedup vs baseline: 7.3991x; 7.3991x over previous
import jax
import jax.numpy as jnp
from jax import lax
from jax.experimental import pallas as pl
from jax.experimental.pallas import tpu as pltpu
from jax.experimental.pallas import tpu_sc as plsc

N = 50000
EN = 800000
C = 64
W = 72
NC, NS, LN = 2, 16, 16
HALF = N // NC
ACC_R = 25200
TRASH = 25000
RPT = ACC_R // NS
CH = 80
EPT = EN // NS
NCHUNK = EPT // CH
CH1 = 400
NCHUNK1 = EPT // CH1


def _side_dense_body(x_ref, w1_ref, wp_ref, wr_ref, b_ref,
                     h_ref, pk_ref, res_ref, st_ref):
    i = pl.program_id(0)
    x = x_ref[...]
    h_ref[...] = jnp.dot(x, w1_ref[...], preferred_element_type=jnp.float32)
    pk = jnp.dot(x, wp_ref[...], preferred_element_type=jnp.float32)
    pk_ref[...] = pk
    res_ref[...] = jnp.dot(x, wr_ref[...], preferred_element_type=jnp.float32) + b_ref[...]
    m = jnp.max(pk[:, 0:1])

    @pl.when(i == 0)
    def _():
        st_ref[...] = jnp.full((8, 128), -jnp.inf, jnp.float32)

    st_ref[...] = jnp.maximum(st_ref[...], m)


def _side_dense(x, w1, wp, wr, bias):
    n, din = x.shape
    br = 1000
    return pl.pallas_call(
        _side_dense_body,
        grid=(n // br,),
        in_specs=[
            pl.BlockSpec((br, din), lambda i: (i, 0)),
            pl.BlockSpec(w1.shape, lambda i: (0, 0)),
            pl.BlockSpec(wp.shape, lambda i: (0, 0)),
            pl.BlockSpec(wr.shape, lambda i: (0, 0)),
            pl.BlockSpec(bias.shape, lambda i: (0, 0)),
        ],
        out_specs=[
            pl.BlockSpec((br, C), lambda i: (i, 0)),
            pl.BlockSpec((br, 8), lambda i: (i, 0)),
            pl.BlockSpec((br, C), lambda i: (i, 0)),
            pl.BlockSpec((8, 128), lambda i: (0, 0)),
        ],
        out_shape=[
            jax.ShapeDtypeStruct((n, C), jnp.float32),
            jax.ShapeDtypeStruct((n, 8), jnp.float32),
            jax.ShapeDtypeStruct((n, C), jnp.float32),
            jax.ShapeDtypeStruct((8, 128), jnp.float32),
        ],
    )(x, w1, wp, wr, bias)


def _edge_dense_body(ea_dt_ref, ea_td_ref, v_dt_ref, v_td_ref,
                     ae_dt_ref, ae_td_ref, s1dt_ref, s2dt_ref, s1td_ref, s2td_ref):
    i = pl.program_id(0)
    a = jnp.dot(ea_dt_ref[...], v_dt_ref[...], preferred_element_type=jnp.float32)
    b = jnp.dot(ea_td_ref[...], v_td_ref[...], preferred_element_type=jnp.float32)
    ae_dt_ref[...] = a
    ae_td_ref[...] = b

    @pl.when(i == 0)
    def _():
        for r in (s1dt_ref, s2dt_ref, s1td_ref, s2td_ref):
            r[...] = jnp.full((8, 128), -jnp.inf, jnp.float32)

    s1dt_ref[...] = jnp.maximum(s1dt_ref[...], jnp.max(a[:, 0:1]))
    s2dt_ref[...] = jnp.maximum(s2dt_ref[...], jnp.max(a[:, 8:9]))
    s1td_ref[...] = jnp.maximum(s1td_ref[...], jnp.max(b[:, 0:1]))
    s2td_ref[...] = jnp.maximum(s2td_ref[...], jnp.max(b[:, 8:9]))


def _edge_dense(ea_dt, ea_td, v_dt, v_td):
    be = 8000
    st = jax.ShapeDtypeStruct((8, 128), jnp.float32)
    stspec = pl.BlockSpec((8, 128), lambda i: (0, 0))
    return pl.pallas_call(
        _edge_dense_body,
        grid=(EN // be,),
        in_specs=[
            pl.BlockSpec((be, 8), lambda i: (i, 0)),
            pl.BlockSpec((be, 8), lambda i: (i, 0)),
            pl.BlockSpec((8, 16), lambda i: (0, 0)),
            pl.BlockSpec((8, 16), lambda i: (0, 0)),
        ],
        out_specs=[
            pl.BlockSpec((be, 16), lambda i: (i, 0)),
            pl.BlockSpec((be, 16), lambda i: (i, 0)),
            stspec, stspec, stspec, stspec,
        ],
        out_shape=[
            jax.ShapeDtypeStruct((EN, 16), jnp.float32),
            jax.ShapeDtypeStruct((EN, 16), jnp.float32),
            st, st, st, st,
        ],
    )(ea_dt, ea_td, v_dt, v_td)


def _fin_body(u_ref, res_ref, y_ref):
    u = u_ref[...]
    y_ref[...] = u[:, 0:C] / (u[:, C:C + 1] + 1e-16) + res_ref[...]


def _fin_ln_body(u_ref, res_ref, g_ref, b_ref, y_ref):
    u = u_ref[...]
    o = u[:, 0:C] / (u[:, C:C + 1] + 1e-16) + res_ref[...]
    o = jnp.maximum(o, 0.01 * o)
    mu = jnp.mean(o, axis=-1, keepdims=True)
    var = jnp.mean((o - mu) * (o - mu), axis=-1, keepdims=True)
    y_ref[...] = (o - mu) * lax.rsqrt(var + 1e-5) * g_ref[...] + b_ref[...]


def _finalize(u, res, g=None, b=None):
    br = 200
    uspec = pl.BlockSpec((br, W), lambda p, i: (p * (ACC_R // br) + i, 0))
    yspec = pl.BlockSpec((br, C), lambda p, i: (p * (HALF // br) + i, 0))
    yshape = jax.ShapeDtypeStruct((N, C), jnp.float32)
    grid = (NC, HALF // br)
    if g is None:
        return pl.pallas_call(
            _fin_body, grid=grid,
            in_specs=[uspec, yspec],
            out_specs=yspec, out_shape=yshape,
        )(u, res)
    gspec = pl.BlockSpec((1, C), lambda p, i: (0, 0))
    return pl.pallas_call(
        _fin_ln_body, grid=grid,
        in_specs=[uspec, yspec, gspec, gspec],
        out_specs=yspec, out_shape=yshape,
    )(u, res, g, b)


def _sc1_body(as_hbm, ad_hbm, ae_hbm, src_hbm, dst_hbm, sta_hbm, ste_hbm,
              ex_hbm, astab, adtab, srcv, dstv, aev, exv, sav, sev):
    c = lax.axis_index("c")
    s = lax.axis_index("s")

    @pl.when(c == 0)
    def _():
        pltpu.sync_copy(as_hbm, astab)
        pltpu.sync_copy(ad_hbm, adtab)
        pltpu.sync_copy(sta_hbm.at[0], sav)
        pltpu.sync_copy(ste_hbm.at[0], sev)
        kvec = sav[pl.ds(0, LN)] + sev[pl.ds(0, LN)]

        def body(i, carry):
            base = s * EPT + i * CH1
            pltpu.sync_copy(src_hbm.at[pl.ds(base, CH1)], srcv)
            pltpu.sync_copy(dst_hbm.at[pl.ds(base, CH1)], dstv)
            pltpu.sync_copy(ae_hbm.at[pl.ds(base, CH1)], aev)
            for g in range(CH1 // LN):
                off = g * LN
                sv = srcv[pl.ds(off, LN)]
                dv = dstv[pl.ds(off, LN)]
                ass = plsc.load_gather(astab, [sv])
                ads = plsc.load_gather(adtab, [dv])
                aes = aev[pl.ds(off, LN)]
                t0 = ass + ads + aes
                alpha = jnp.maximum(t0, 0.2 * t0)
                cd = ads + kvec
                cc = jnp.maximum(cd, 0.2 * cd)
                exv[pl.ds(off, LN)] = jnp.exp(alpha - cc)
            pltpu.sync_copy(exv, ex_hbm.at[pl.ds(base, CH1)])
            return carry

        lax.fori_loop(0, NCHUNK1, body, 0)


def _sc2_body(h_hbm, ex_hbm, src_hbm, dst_hbm, u_hbm,
              srcv, dstv, locv0, locv1, exv, rin, rout0, rout1, acc, sem):
    c = lax.axis_index("c")
    s = lax.axis_index("s")
    zero = jnp.zeros((LN,), jnp.float32)
    iota = lax.iota(jnp.int32, LN)
    colc = jnp.full((LN,), C, jnp.int32)
    for r in range(CH):
        for q in range(C // LN):
            rout0[r, pl.ds(q * LN, LN)] = zero
    for g in range(CH // LN):
        plsc.store_scatter(rout0, [iota + g * LN, colc], zero)
    for r in range(RPT // 75):
        pltpu.sync_copy(rout0.at[pl.ds(0, 75)],
                        acc.at[pl.ds(s * RPT + r * 75, 75)])
    plsc.subcore_barrier()
    c0 = c * HALF

    def chunk_step(i, rout, locv):
        base = s * EPT + i * CH
        pltpu.sync_copy(src_hbm.at[pl.ds(base, CH)], srcv)
        pltpu.sync_copy(dst_hbm.at[pl.ds(base, CH)], dstv)
        pltpu.sync_copy(ex_hbm.at[pl.ds(base, CH)], exv)
        cp = pltpu.async_copy(h_hbm.at[srcv], rin, sem)
        for g in range(CH // LN):
            off = g * LN
            dv = dstv[pl.ds(off, LN)]
            loc = dv - c0
            ok = (loc >= 0) & (loc < HALF)
            locv[pl.ds(off, LN)] = jnp.where(ok, loc, TRASH)
            plsc.store_scatter(rout, [iota + off, colc], exv[pl.ds(off, LN)])
        cp.wait()
        for g in range(CH // LN):
            exs = exv[pl.ds(g * LN, LN)]
            for j in range(LN):
                e = g * LN + j
                exb = jnp.full((LN,), exs[j], jnp.float32)
                for q in range(C // LN):
                    rout[e, pl.ds(q * LN, LN)] = rin[e, pl.ds(q * LN, LN)] * exb
        pltpu.sync_copy(rout, acc.at[locv], add=True)

    def body(p, carry):
        chunk_step(2 * p, rout0, locv0)
        chunk_step(2 * p + 1, rout1, locv1)
        return carry

    lax.fori_loop(0, NCHUNK // 2, body, 0)
    chunk_step(NCHUNK - 1, rout0, locv0)
    plsc.subcore_barrier()
    pltpu.sync_copy(acc.at[pl.ds(s * RPT, RPT)],
                    u_hbm.at[pl.ds(c * ACC_R + s * RPT, RPT)])


_SC_CACHE = {}


def _sc_mesh():
    return plsc.VectorSubcoreMesh(
        core_axis_name="c", subcore_axis_name="s",
        num_cores=NC, num_subcores=NS)


def _sc_compiler_params():
    return pltpu.CompilerParams(
        use_tc_tiling_on_sc=False, needs_layout_passes=False)


def _sc_scalar(*args):
    if "k1" not in _SC_CACHE:
        _SC_CACHE["k1"] = pl.kernel(
            _sc1_body, mesh=_sc_mesh(),
            out_type=jax.ShapeDtypeStruct((EN,), jnp.float32),
            compiler_params=_sc_compiler_params(),
            scratch_types=[
                pltpu.VMEM((N,), jnp.float32),
                pltpu.VMEM((N,), jnp.float32),
                pltpu.VMEM((CH1,), jnp.int32),
                pltpu.VMEM((CH1,), jnp.int32),
                pltpu.VMEM((CH1,), jnp.float32),
                pltpu.VMEM((CH1,), jnp.float32),
                pltpu.VMEM((128,), jnp.float32),
                pltpu.VMEM((128,), jnp.float32),
            ])
    return _SC_CACHE["k1"](*args)


def _sc_heavy(*args):
    if "k2" not in _SC_CACHE:
        _SC_CACHE["k2"] = pl.kernel(
            _sc2_body, mesh=_sc_mesh(),
            out_type=jax.ShapeDtypeStruct((NC * ACC_R, W), jnp.float32),
            compiler_params=_sc_compiler_params(),
            scratch_types=[
                pltpu.VMEM((CH,), jnp.int32),
                pltpu.VMEM((CH,), jnp.int32),
                pltpu.VMEM((CH,), jnp.int32),
                pltpu.VMEM((CH,), jnp.int32),
                pltpu.VMEM((CH,), jnp.float32),
                pltpu.VMEM((CH, C), jnp.float32),
                pltpu.VMEM((CH, W), jnp.float32),
                pltpu.VMEM((CH, W), jnp.float32),
                pltpu.VMEM_SHARED((ACC_R, W), jnp.float32),
                pltpu.SemaphoreType.DMA,
            ])
    return _SC_CACHE["k2"](*args)


def kernel(x_tasks, x_data, edge_index_dt, edge_index_td,
           edge_attr_dt, edge_attr_td, params):
    f32 = jnp.float32
    p1dt, p1td = params["c1_dt"], params["c1_td"]
    p2dt, p2td = params["c2_dt"], params["c2_td"]

    def pad_rows(w, din, pin):
        return jnp.zeros((pin, w.shape[1]), f32).at[:din].set(w)

    def packed_weights(p_as_src, p_as_dst, din, pin):
        w1 = pad_rows(p_as_src["w_src"], din, pin)
        v_as = p_as_src["w_src"] @ p_as_src["att_src"].reshape(C)
        v_ad = p_as_dst["w_dst"] @ p_as_dst["att_dst"].reshape(C)
        wp = jnp.zeros((pin, 8), f32)
        wp = wp.at[:din, 0].set(v_as).at[:din, 1].set(v_ad)
        wr = pad_rows(p_as_dst["w_res"], din, pin)
        bias = p_as_dst["bias"].reshape(1, C)
        return w1, wp, wr, bias

    def pad_x(x, pin):
        n, din = x.shape
        return jnp.zeros((n, pin), f32).at[:, :din].set(x)

    def ve16(p_dir1, p_dir2):
        v1 = p_dir1["w_edge"] @ p_dir1["att_edge"].reshape(C)
        v2 = p_dir2["w_edge"] @ p_dir2["att_edge"].reshape(C)
        v = jnp.zeros((8, 16), f32)
        v = v.at[:3, 0:8].set(jnp.tile(v1[:, None], (1, 8)))
        v = v.at[:3, 8:16].set(jnp.tile(v2[:, None], (1, 8)))
        return v

    ea_dt8 = pad_x(edge_attr_dt, 8)
    ea_td8 = pad_x(edge_attr_td, 8)
    ae_dt, ae_td, s1dt_e, s2dt_e, s1td_e, s2td_e = _edge_dense(
        ea_dt8, ea_td8, ve16(p1dt, p2dt), ve16(p1td, p2td))
    ae1_dt, ae2_dt = ae_dt[:, 0], ae_dt[:, 8]
    ae1_td, ae2_td = ae_td[:, 0], ae_td[:, 8]

    src_dt, dst_dt = edge_index_dt[0], edge_index_dt[1]
    src_td, dst_td = edge_index_td[0], edge_index_td[1]

    h_d, pk_d, res_td, st_d = _side_dense(
        pad_x(x_data, 8), *packed_weights(p1dt, p1td, 5, 8))
    h_t, pk_t, res_dt, st_t = _side_dense(
        pad_x(x_tasks, 16), *packed_weights(p1td, p1dt, 12, 16))

    ex_dt = _sc_scalar(pk_d[:, 0], pk_t[:, 1], ae1_dt, src_dt, dst_dt,
                       st_d, s1dt_e)
    u_dt = _sc_heavy(h_d, ex_dt, src_dt, dst_dt)
    ex_td = _sc_scalar(pk_t[:, 0], pk_d[:, 1], ae1_td, src_td, dst_td,
                       st_t, s1td_e)
    u_td = _sc_heavy(h_t, ex_td, src_td, dst_td)

    t1 = _finalize(u_dt, res_dt, params["ln_t_g"].reshape(1, C),
                   params["ln_t_b"].reshape(1, C))
    d1 = _finalize(u_td, res_td, params["ln_d_g"].reshape(1, C),
                   params["ln_d_b"].reshape(1, C))

    h_d2, pk_d2, res_td2, st_d2 = _side_dense(
        d1, *packed_weights(p2dt, p2td, C, C))
    h_t2, pk_t2, res_dt2, st_t2 = _side_dense(
        t1, *packed_weights(p2td, p2dt, C, C))

    ex_dt2 = _sc_scalar(pk_d2[:, 0], pk_t2[:, 1], ae2_dt, src_dt, dst_dt,
                        st_d2, s2dt_e)
    u_dt2 = _sc_heavy(h_d2, ex_dt2, src_dt, dst_dt)
    ex_td2 = _sc_scalar(pk_t2[:, 0], pk_d2[:, 1], ae2_td, src_td, dst_td,
                        st_t2, s2td_e)
    u_td2 = _sc_heavy(h_t2, ex_td2, src_td, dst_td)

    t2 = _finalize(u_dt2, res_dt2)
    d2 = _finalize(u_td2, res_td2)
    return (t2, d2)


# speedup vs baseline: 8.4205x; 1.1380x over previous
import functools

import jax
import jax.numpy as jnp
from jax import lax
from jax.experimental import pallas as pl
from jax.experimental.pallas import tpu as pltpu
from jax.experimental.pallas import tpu_sc as plsc

N = 50000
EN = 800000
C = 64
W = 72
NC, NS, LN = 2, 16, 16
HALF = N // NC
ACC_R = 25200
TRASH = 25000
RPT = ACC_R // NS
CH = 80
EPT = EN // NS
NCHUNK = EPT // CH
CH1 = 400
NCHUNK1 = EPT // CH1


def _side_dense_body(x_ref, w1_ref, wp_ref, wr_ref, b_ref,
                     h_ref, pk_ref, res_ref, st_ref):
    i = pl.program_id(0)
    x = x_ref[...]
    h_ref[...] = jnp.dot(x, w1_ref[...], preferred_element_type=jnp.float32)
    pk = jnp.dot(x, wp_ref[...], preferred_element_type=jnp.float32)
    pk_ref[...] = pk
    res_ref[...] = jnp.dot(x, wr_ref[...], preferred_element_type=jnp.float32) + b_ref[...]
    m = jnp.max(pk[:, 0:1])

    @pl.when(i == 0)
    def _():
        st_ref[...] = jnp.full((8, 128), -jnp.inf, jnp.float32)

    st_ref[...] = jnp.maximum(st_ref[...], m)


def _side_dense(x, w1, wp, wr, bias):
    n, din = x.shape
    br = 1000
    return pl.pallas_call(
        _side_dense_body,
        grid=(n // br,),
        in_specs=[
            pl.BlockSpec((br, din), lambda i: (i, 0)),
            pl.BlockSpec(w1.shape, lambda i: (0, 0)),
            pl.BlockSpec(wp.shape, lambda i: (0, 0)),
            pl.BlockSpec(wr.shape, lambda i: (0, 0)),
            pl.BlockSpec(bias.shape, lambda i: (0, 0)),
        ],
        out_specs=[
            pl.BlockSpec((br, C), lambda i: (i, 0)),
            pl.BlockSpec((br, 8), lambda i: (i, 0)),
            pl.BlockSpec((br, C), lambda i: (i, 0)),
            pl.BlockSpec((8, 128), lambda i: (0, 0)),
        ],
        out_shape=[
            jax.ShapeDtypeStruct((n, C), jnp.float32),
            jax.ShapeDtypeStruct((n, 8), jnp.float32),
            jax.ShapeDtypeStruct((n, C), jnp.float32),
            jax.ShapeDtypeStruct((8, 128), jnp.float32),
        ],
    )(x, w1, wp, wr, bias)


def _edge_dense_body(ea_dt_ref, ea_td_ref, v_dt_ref, v_td_ref,
                     ae_dt_ref, ae_td_ref, s1dt_ref, s2dt_ref, s1td_ref, s2td_ref):
    i = pl.program_id(0)
    a = jnp.dot(ea_dt_ref[...], v_dt_ref[...], preferred_element_type=jnp.float32)
    b = jnp.dot(ea_td_ref[...], v_td_ref[...], preferred_element_type=jnp.float32)
    ae_dt_ref[...] = a
    ae_td_ref[...] = b

    @pl.when(i == 0)
    def _():
        for r in (s1dt_ref, s2dt_ref, s1td_ref, s2td_ref):
            r[...] = jnp.full((8, 128), -jnp.inf, jnp.float32)

    s1dt_ref[...] = jnp.maximum(s1dt_ref[...], jnp.max(a[:, 0:1]))
    s2dt_ref[...] = jnp.maximum(s2dt_ref[...], jnp.max(a[:, 8:9]))
    s1td_ref[...] = jnp.maximum(s1td_ref[...], jnp.max(b[:, 0:1]))
    s2td_ref[...] = jnp.maximum(s2td_ref[...], jnp.max(b[:, 8:9]))


def _edge_dense(ea_dt, ea_td, v_dt, v_td):
    be = 8000
    st = jax.ShapeDtypeStruct((8, 128), jnp.float32)
    stspec = pl.BlockSpec((8, 128), lambda i: (0, 0))
    return pl.pallas_call(
        _edge_dense_body,
        grid=(EN // be,),
        in_specs=[
            pl.BlockSpec((be, 8), lambda i: (i, 0)),
            pl.BlockSpec((be, 8), lambda i: (i, 0)),
            pl.BlockSpec((8, 16), lambda i: (0, 0)),
            pl.BlockSpec((8, 16), lambda i: (0, 0)),
        ],
        out_specs=[
            pl.BlockSpec((be, 16), lambda i: (i, 0)),
            pl.BlockSpec((be, 16), lambda i: (i, 0)),
            stspec, stspec, stspec, stspec,
        ],
        out_shape=[
            jax.ShapeDtypeStruct((EN, 16), jnp.float32),
            jax.ShapeDtypeStruct((EN, 16), jnp.float32),
            st, st, st, st,
        ],
    )(ea_dt, ea_td, v_dt, v_td)


def _fin_body(u_ref, res_ref, y_ref):
    u = u_ref[...]
    y_ref[...] = u[:, 0:C] / (u[:, C:C + 1] + 1e-16) + res_ref[...]


def _fin_ln_body(u_ref, res_ref, g_ref, b_ref, y_ref):
    u = u_ref[...]
    o = u[:, 0:C] / (u[:, C:C + 1] + 1e-16) + res_ref[...]
    o = jnp.maximum(o, 0.01 * o)
    mu = jnp.mean(o, axis=-1, keepdims=True)
    var = jnp.mean((o - mu) * (o - mu), axis=-1, keepdims=True)
    y_ref[...] = (o - mu) * lax.rsqrt(var + 1e-5) * g_ref[...] + b_ref[...]


def _finalize(u, res, g=None, b=None):
    br = 200
    uspec = pl.BlockSpec((br, W), lambda p, i: (p * (ACC_R // br) + i, 0))
    yspec = pl.BlockSpec((br, C), lambda p, i: (p * (HALF // br) + i, 0))
    yshape = jax.ShapeDtypeStruct((N, C), jnp.float32)
    grid = (NC, HALF // br)
    if g is None:
        return pl.pallas_call(
            _fin_body, grid=grid,
            in_specs=[uspec, yspec],
            out_specs=yspec, out_shape=yshape,
        )(u, res)
    gspec = pl.BlockSpec((1, C), lambda p, i: (0, 0))
    return pl.pallas_call(
        _fin_ln_body, grid=grid,
        in_specs=[uspec, yspec, gspec, gspec],
        out_specs=yspec, out_shape=yshape,
    )(u, res, g, b)


def _sc_body(ae_col, h_hbm, pks_hbm, pkd_hbm, ae_hbm, src_hbm, dst_hbm,
             sta_hbm, ste_hbm, u_hbm, srcv, dstv, locv, asr, adr, aev,
             sav, sev, rin, rout, acc, semh, semp, semq):
    c = lax.axis_index("c")
    s = lax.axis_index("s")
    zero = jnp.zeros((LN,), jnp.float32)
    iota = lax.iota(jnp.int32, LN)
    colc = jnp.full((LN,), C, jnp.int32)
    for r in range(CH):
        for q in range(C // LN):
            rout[r, pl.ds(q * LN, LN)] = zero
    for g in range(CH // LN):
        plsc.store_scatter(rout, [iota + g * LN, colc], zero)
    for r in range(RPT // 75):
        pltpu.sync_copy(rout.at[pl.ds(0, 75)],
                        acc.at[pl.ds(s * RPT + r * 75, 75)])
    pltpu.sync_copy(sta_hbm.at[0], sav)
    pltpu.sync_copy(ste_hbm.at[0], sev)
    plsc.subcore_barrier()
    kvec = sav[pl.ds(0, LN)] + sev[pl.ds(0, LN)]
    c0 = c * HALF

    def chunk_step(i, carry):
        base = s * EPT + i * CH
        pltpu.sync_copy(src_hbm.at[pl.ds(base, CH)], srcv)
        pltpu.sync_copy(dst_hbm.at[pl.ds(base, CH)], dstv)
        cph = pltpu.async_copy(h_hbm.at[srcv], rin, semh)
        cpa = pltpu.async_copy(pks_hbm.at[srcv], asr, semp)
        cpb = pltpu.async_copy(pkd_hbm.at[dstv], adr, semq)
        pltpu.sync_copy(ae_hbm.at[pl.ds(base, CH)], aev)
        zv = srcv[pl.ds(0, LN)] * 0
        cpa.wait()
        cpb.wait()
        cph.wait()
        for g in range(CH // LN):
            off = g * LN
            dv = dstv[pl.ds(off, LN)]
            ass = plsc.load_gather(asr, [iota + off, zv])
            ads = plsc.load_gather(adr, [iota + off, zv + 1])
            aes = plsc.load_gather(aev, [iota + off, zv + ae_col])
            t0 = ass + ads + aes
            alpha = jnp.maximum(t0, 0.2 * t0)
            cd = ads + kvec
            cc = jnp.maximum(cd, 0.2 * cd)
            ex = jnp.exp(alpha - cc)
            loc = dv - c0
            ok = (loc >= 0) & (loc < HALF)
            locv[pl.ds(off, LN)] = jnp.where(ok, loc, TRASH)
            plsc.store_scatter(rout, [iota + off, colc], ex)
            for j in range(LN):
                e = off + j
                exb = jnp.full((LN,), ex[j], jnp.float32)
                for q in range(C // LN):
                    rout[e, pl.ds(q * LN, LN)] = rin[e, pl.ds(q * LN, LN)] * exb
        pltpu.sync_copy(rout, acc.at[locv], add=True)
        return carry

    lax.fori_loop(0, NCHUNK, chunk_step, 0)
    plsc.subcore_barrier()
    pltpu.sync_copy(acc.at[pl.ds(s * RPT, RPT)],
                    u_hbm.at[pl.ds(c * ACC_R + s * RPT, RPT)])


_SC_CACHE = {}


def _sc_mesh():
    return plsc.VectorSubcoreMesh(
        core_axis_name="c", subcore_axis_name="s",
        num_cores=NC, num_subcores=NS)


def _sc_compiler_params():
    return pltpu.CompilerParams(
        use_tc_tiling_on_sc=False, needs_layout_passes=False)


def _sc_gat(ae_col, *args):
    key = ("gat", ae_col)
    if key not in _SC_CACHE:
        _SC_CACHE[key] = pl.kernel(
            functools.partial(_sc_body, ae_col), mesh=_sc_mesh(),
            out_type=jax.ShapeDtypeStruct((NC * ACC_R, W), jnp.float32),
            compiler_params=_sc_compiler_params(),
            scratch_types=[
                pltpu.VMEM((CH,), jnp.int32),
                pltpu.VMEM((CH,), jnp.int32),
                pltpu.VMEM((CH,), jnp.int32),
                pltpu.VMEM((CH, 8), jnp.float32),
                pltpu.VMEM((CH, 8), jnp.float32),
                pltpu.VMEM((CH, 16), jnp.float32),
                pltpu.VMEM((128,), jnp.float32),
                pltpu.VMEM((128,), jnp.float32),
                pltpu.VMEM((CH, C), jnp.float32),
                pltpu.VMEM((CH, W), jnp.float32),
                pltpu.VMEM_SHARED((ACC_R, W), jnp.float32),
                pltpu.SemaphoreType.DMA,
                pltpu.SemaphoreType.DMA,
                pltpu.SemaphoreType.DMA,
            ])
    return _SC_CACHE[key](*args)


def kernel(x_tasks, x_data, edge_index_dt, edge_index_td,
           edge_attr_dt, edge_attr_td, params):
    f32 = jnp.float32
    p1dt, p1td = params["c1_dt"], params["c1_td"]
    p2dt, p2td = params["c2_dt"], params["c2_td"]

    def pad_rows(w, din, pin):
        return jnp.zeros((pin, w.shape[1]), f32).at[:din].set(w)

    def packed_weights(p_as_src, p_as_dst, din, pin):
        w1 = pad_rows(p_as_src["w_src"], din, pin)
        v_as = p_as_src["w_src"] @ p_as_src["att_src"].reshape(C)
        v_ad = p_as_dst["w_dst"] @ p_as_dst["att_dst"].reshape(C)
        wp = jnp.zeros((pin, 8), f32)
        wp = wp.at[:din, 0].set(v_as).at[:din, 1].set(v_ad)
        wr = pad_rows(p_as_dst["w_res"], din, pin)
        bias = p_as_dst["bias"].reshape(1, C)
        return w1, wp, wr, bias

    def pad_x(x, pin):
        n, din = x.shape
        return jnp.zeros((n, pin), f32).at[:, :din].set(x)

    def ve16(p_dir1, p_dir2):
        v1 = p_dir1["w_edge"] @ p_dir1["att_edge"].reshape(C)
        v2 = p_dir2["w_edge"] @ p_dir2["att_edge"].reshape(C)
        v = jnp.zeros((8, 16), f32)
        v = v.at[:3, 0:8].set(jnp.tile(v1[:, None], (1, 8)))
        v = v.at[:3, 8:16].set(jnp.tile(v2[:, None], (1, 8)))
        return v

    ea_dt8 = pad_x(edge_attr_dt, 8)
    ea_td8 = pad_x(edge_attr_td, 8)
    ae_dt, ae_td, s1dt_e, s2dt_e, s1td_e, s2td_e = _edge_dense(
        ea_dt8, ea_td8, ve16(p1dt, p2dt), ve16(p1td, p2td))

    src_dt, dst_dt = edge_index_dt[0], edge_index_dt[1]
    src_td, dst_td = edge_index_td[0], edge_index_td[1]

    h_d, pk_d, res_td, st_d = _side_dense(
        pad_x(x_data, 8), *packed_weights(p1dt, p1td, 5, 8))
    h_t, pk_t, res_dt, st_t = _side_dense(
        pad_x(x_tasks, 16), *packed_weights(p1td, p1dt, 12, 16))

    u_dt = _sc_gat(0, h_d, pk_d, pk_t, ae_dt, src_dt, dst_dt, st_d, s1dt_e)
    u_td = _sc_gat(0, h_t, pk_t, pk_d, ae_td, src_td, dst_td, st_t, s1td_e)

    t1 = _finalize(u_dt, res_dt, params["ln_t_g"].reshape(1, C),
                   params["ln_t_b"].reshape(1, C))
    d1 = _finalize(u_td, res_td, params["ln_d_g"].reshape(1, C),
                   params["ln_d_b"].reshape(1, C))

    h_d2, pk_d2, res_td2, st_d2 = _side_dense(
        d1, *packed_weights(p2dt, p2td, C, C))
    h_t2, pk_t2, res_dt2, st_t2 = _side_dense(
        t1, *packed_weights(p2td, p2dt, C, C))

    u_dt2 = _sc_gat(8, h_d2, pk_d2, pk_t2, ae_dt, src_dt, dst_dt,
                    st_d2, s2dt_e)
    u_td2 = _sc_gat(8, h_t2, pk_t2, pk_d2, ae_td, src_td, dst_td,
                    st_t2, s2td_e)

    t2 = _finalize(u_dt2, res_dt2)
    d2 = _finalize(u_td2, res_td2)
    return (t2, d2)
